# Initial kernel scaffold; baseline (speedup 1.0000x reference)
import jax
import jax.numpy as jnp
from jax.experimental import pallas as pl


def kernel(X, edge_index, batch, W1, b1, W2, b2, W3, b3, W4, b4, W5, b5, g1, be1, g2, be2, g3, be3, g4, be4, g5, be5, lin_W, lin_b):
    raise NotImplementedError("write your pallas kernel here")


# speedup vs baseline: 7.9613x; 7.9613x over previous
import dataclasses
import functools

import jax
import jax.numpy as jnp
from jax import lax
from jax.experimental import pallas as pl
from jax.experimental.pallas import tpu as pltpu
from jax.experimental.pallas import tpu_sc as plsc

N = 10000
E = 320000
D = 128
G = 64
C = 10

NC = 2
NS = 16
NW = NC * NS
B = 128
NBLK = -(-E // (NW * B)) * NW
PE = NBLK * B
BPW = NBLK // NW
P = 10240
RPW = P // NS

_HIGH = lax.Precision.HIGHEST


def _mesh():
    return plsc.VectorSubcoreMesh(core_axis_name="c", subcore_axis_name="s")


def _sc_params():
    cp = pltpu.CompilerParams()
    if "needs_layout_passes" in pltpu.CompilerParams.__dataclass_fields__:
        cp = dataclasses.replace(cp, needs_layout_passes=False)
    return cp


_TC_PARAMS = pltpu.CompilerParams(vmem_limit_bytes=64 * 1024 * 1024)


def _sc_degree(dstb):
    @functools.partial(
        pl.kernel,
        out_type=jax.ShapeDtypeStruct((NW, P), jnp.float32),
        mesh=_mesh(),
        compiler_params=_sc_params(),
        scratch_types=[
            pltpu.VMEM((1, B), jnp.int32),
            pltpu.VMEM((P,), jnp.float32),
        ],
    )
    def k(dst_hbm, out_hbm, dbuf, hist):
        c = lax.axis_index("c")
        s = lax.axis_index("s")
        w = s * NC + c

        zeros = jnp.zeros((16,), jnp.float32)

        @pl.loop(0, P // 16)
        def _(i):
            hist[pl.ds(i * 16, 16)] = zeros

        ones = jnp.ones((16,), jnp.float32)

        @pl.loop(0, BPW)
        def _(i):
            pltpu.sync_copy(dst_hbm.at[pl.ds(w * BPW + i, 1)], dbuf)

            @pl.loop(0, B // 16)
            def _(j):
                idx = dbuf[0, pl.ds(j * 16, 16)]
                plsc.addupdate_scatter(hist, [idx], ones)

        pltpu.sync_copy(hist, out_hbm.at[w])

    return k(dstb)


def _sc_scatter(hp, srcb, dstb):
    @functools.partial(
        pl.kernel,
        out_type=jax.ShapeDtypeStruct((NC, P, D), jnp.float32),
        mesh=_mesh(),
        compiler_params=_sc_params(),
        scratch_types=[
            pltpu.VMEM((1, B), jnp.int32),
            pltpu.VMEM((1, B), jnp.int32),
            pltpu.VMEM((B, D), jnp.float32),
            pltpu.VMEM_SHARED((P, D), jnp.float32),
        ],
    )
    def k(hp_hbm, src_hbm, dst_hbm, out_hbm, sbuf, dbuf, rows, acc):
        c = lax.axis_index("c")
        s = lax.axis_index("s")
        w = s * NC + c

        pltpu.sync_copy(hp_hbm.at[pl.ds(s * RPW, RPW)],
                        acc.at[pl.ds(s * RPW, RPW)])
        plsc.subcore_barrier()

        @pl.loop(0, BPW)
        def _(i):
            blk = w * BPW + i
            pltpu.sync_copy(src_hbm.at[pl.ds(blk, 1)], sbuf)
            pltpu.sync_copy(dst_hbm.at[pl.ds(blk, 1)], dbuf)
            pltpu.sync_copy(hp_hbm.at[sbuf.at[0]], rows)
            pltpu.sync_copy(rows, acc.at[dbuf.at[0]], add=True)

        plsc.subcore_barrier()
        pltpu.sync_copy(acc.at[pl.ds(s * RPW, RPW)],
                        out_hbm.at[c, pl.ds(s * RPW, RPW)])

    return k(hp, srcb, dstb)


def _tc_dinv(degp):
    def body(p_ref, o_ref):
        deg = jnp.sum(p_ref[...], axis=0) + 1.0
        o_ref[...] = lax.rsqrt(deg)

    return pl.pallas_call(
        body, out_shape=jax.ShapeDtypeStruct((P,), jnp.float32))(degp)


def _tc_pre(xp, w1, dinv_c):
    def body(x_ref, w_ref, dv_ref, o_ref):
        h = jnp.dot(x_ref[...], w_ref[...],
                    preferred_element_type=jnp.float32, precision=_HIGH)
        o_ref[...] = h * dv_ref[...]

    return pl.pallas_call(
        body, out_shape=jax.ShapeDtypeStruct((P, D), jnp.float32))(
            xp, w1, dinv_c)


def _post_agg(p0, p1, hp, dv, b, g, be, res):
    agg = (p0 + p1 - hp) * dv + b
    t = jnp.maximum(agg, 0.0)
    mask = lax.broadcasted_iota(jnp.int32, (P, 1), 0) < N
    t = jnp.where(mask, t, 0.0)
    m = jnp.sum(t, axis=0) / N
    v = jnp.sum(t * t, axis=0) / N - m * m
    y = (t - m) * lax.rsqrt(v + 1e-5) * g + be
    if res is not None:
        y = y + res
    return jnp.where(mask, y, 0.0)


def _tc_mid(parts, hp, dinv_c, b, g, be, res, w_next):
    with_res = res is not None

    def body(*refs):
        if with_res:
            (p_ref, hp_ref, dv_ref, b_ref, g_ref, be_ref, res_ref, w_ref,
             x_out, hpn_out) = refs
            res_v = res_ref[...]
        else:
            (p_ref, hp_ref, dv_ref, b_ref, g_ref, be_ref, w_ref,
             x_out, hpn_out) = refs
            res_v = None
        y = _post_agg(p_ref[0], p_ref[1], hp_ref[...], dv_ref[...],
                      b_ref[...], g_ref[...], be_ref[...], res_v)
        x_out[...] = y
        hpn = jnp.dot(y, w_ref[...],
                      preferred_element_type=jnp.float32, precision=_HIGH)
        hpn_out[...] = hpn * dv_ref[...]

    args = [parts, hp, dinv_c, b, g, be] + ([res] if with_res else []) + [w_next]
    return pl.pallas_call(
        body,
        out_shape=[jax.ShapeDtypeStruct((P, D), jnp.float32),
                   jax.ShapeDtypeStruct((P, D), jnp.float32)],
        compiler_params=_TC_PARAMS)(*args)


def _tc_final(parts, hp, dinv_c, b, g, be, res, batch_p, lin_w, lin_b):
    def body(p_ref, hp_ref, dv_ref, b_ref, g_ref, be_ref, res_ref,
             bt_ref, lw_ref, lb_ref, o_ref):
        y = _post_agg(p_ref[0], p_ref[1], hp_ref[...], dv_ref[...],
                      b_ref[...], g_ref[...], be_ref[...], res_ref[...])
        seg = lax.broadcasted_iota(jnp.int32, (G, P), 0)
        msk = (bt_ref[...][None, :] == seg).astype(jnp.float32)
        sums = jnp.dot(msk, y, preferred_element_type=jnp.float32,
                       precision=_HIGH)
        cnt = jnp.sum(msk, axis=1, keepdims=True)
        pooled = sums / jnp.maximum(cnt, 1.0)
        logits = jnp.dot(pooled, lw_ref[...],
                         preferred_element_type=jnp.float32,
                         precision=_HIGH) + lb_ref[...]
        o_ref[...] = jax.nn.softmax(logits, axis=-1)

    return pl.pallas_call(
        body, out_shape=jax.ShapeDtypeStruct((G, C), jnp.float32),
        compiler_params=_TC_PARAMS)(
            parts, hp, dinv_c, b, g, be, res, batch_p, lin_w, lin_b)


def kernel(X, edge_index, batch, W1, b1, W2, b2, W3, b3, W4, b4, W5, b5,
           g1, be1, g2, be2, g3, be3, g4, be4, g5, be5, lin_W, lin_b):
    pad_idx = jnp.full((PE - E,), N, jnp.int32)
    srcb = jnp.concatenate([edge_index[0], pad_idx]).reshape(NBLK, B)
    dstb = jnp.concatenate([edge_index[1], pad_idx]).reshape(NBLK, B)
    xp = jnp.pad(X, ((0, P - N), (0, 0)))
    batch_p = jnp.concatenate([batch, jnp.full((P - N,), G, jnp.int32)])

    degp = _sc_degree(dstb)
    dinv = _tc_dinv(degp)
    dinv_c = dinv.reshape(P, 1)

    hp = _tc_pre(xp, W1, dinv_c)

    bs = (b1, b2, b3, b4)
    gs = (g1, g2, g3, g4)
    bes = (be1, be2, be3, be4)
    wn = (W2, W3, W4, W5)
    x_prev = None
    for i in range(4):
        parts = _sc_scatter(hp, srcb, dstb)
        x_prev, hp = _tc_mid(parts, hp, dinv_c, bs[i], gs[i], bes[i],
                             x_prev, wn[i])

    parts = _sc_scatter(hp, srcb, dstb)
    return _tc_final(parts, hp, dinv_c, b5, g5, be5, x_prev, batch_p,
                     lin_W, lin_b)
